# baseline (speedup 1.0000x reference)
import functools

import jax
import jax.numpy as jnp
from jax import lax
from jax.experimental import pallas as pl
from jax.experimental.pallas import tpu as pltpu
from jax.experimental.pallas import tpu_sc as plsc

N = 10000
E = 320000
D = 128
L_LAYERS = 4

NC = 2
NS = 16
NW = NC * NS

N_PAD = 10240
NPG = 2 * N_PAD
ROWS_PER_W = NPG // NW
K_IDX = 5
CHUNK = K_IDX * 128
E_PAD = 327680
EDGES_PER_SUB = E_PAD // NS
IDX_SLAB = 16
N_SLABS = EDGES_PER_SUB // (IDX_SLAB * 128)

_MESH = plsc.VectorSubcoreMesh(
    core_axis_name="c", subcore_axis_name="s", num_cores=NC, num_subcores=NS)


@functools.partial(
    pl.kernel,
    out_type=jax.ShapeDtypeStruct((NPG, D), jnp.float32),
    mesh=_MESH,
    scratch_types=[
        pltpu.VMEM((NPG // 128, 128), jnp.int32),
        pltpu.VMEM((CHUNK, D), jnp.float32),
        pltpu.SemaphoreType.DMA,
    ],
)
def _sc_embed_gather(tab_hbm, idx_hbm, out_hbm, idx_v, rows_v, sem):
    wid = lax.axis_index("s") * NC + lax.axis_index("c")
    pltpu.sync_copy(idx_hbm, idx_v)
    descs = []
    for j in range(K_IDX):
        descs.append(pltpu.async_copy(
            tab_hbm.at[idx_v.at[wid * K_IDX + j]],
            rows_v.at[pl.ds(j * 128, 128)], sem))
    for d in descs:
        d.wait()
    pltpu.sync_copy(rows_v, out_hbm.at[pl.ds(wid * ROWS_PER_W, ROWS_PER_W)])


@functools.partial(
    pl.kernel,
    out_type=jax.ShapeDtypeStruct((NPG, D), jnp.float32),
    mesh=_MESH,
    scratch_types=[
        pltpu.VMEM((IDX_SLAB, 128), jnp.int32),
        pltpu.VMEM((IDX_SLAB, 128), jnp.int32),
        pltpu.VMEM((256, D), jnp.float32),
        pltpu.VMEM_SHARED((N_PAD, D), jnp.float32),
        pltpu.SemaphoreType.DMA,
        pltpu.SemaphoreType.DMA,
    ],
)
def _sc_edge_aggregate(m_hbm, src_hbm, dst_hbm, zeros_hbm, agg_hbm,
                       src_v, dst_v, rows_v, accum, sem, sem2):
    c = lax.axis_index("c")
    s = lax.axis_index("s")
    rows_per_sub = N_PAD // NS
    pltpu.sync_copy(zeros_hbm, rows_v)
    for off, ln in ((0, 256), (256, 256), (512, 128)):
        pltpu.sync_copy(rows_v.at[pl.ds(0, ln)],
                        accum.at[pl.ds(s * rows_per_sub + off, ln)])
    plsc.subcore_barrier()

    idx_base = c * (E_PAD // 128) + s * (EDGES_PER_SUB // 128)

    def slab_body(t, carry):
        pltpu.sync_copy(src_hbm.at[pl.ds(idx_base + t * IDX_SLAB, IDX_SLAB)],
                        src_v)
        pltpu.sync_copy(dst_hbm.at[pl.ds(idx_base + t * IDX_SLAB, IDX_SLAB)],
                        dst_v)
        descs = [None, None]
        descs[0] = pltpu.async_copy(
            m_hbm.at[src_v.at[0]], rows_v.at[pl.ds(0, 128)], sem)
        for j in range(IDX_SLAB):
            if j + 1 < IDX_SLAB:
                descs[(j + 1) % 2] = pltpu.async_copy(
                    m_hbm.at[src_v.at[j + 1]],
                    rows_v.at[pl.ds(((j + 1) % 2) * 128, 128)], sem)
            descs[j % 2].wait()
            pltpu.sync_copy(rows_v.at[pl.ds((j % 2) * 128, 128)],
                            accum.at[dst_v.at[j]], add=True)
        return carry

    lax.fori_loop(0, N_SLABS, slab_body, 0)
    plsc.subcore_barrier()
    for off, ln in ((0, 256), (256, 256), (512, 128)):
        pltpu.sync_copy(accum.at[pl.ds(s * rows_per_sub + off, ln)],
                        rows_v.at[pl.ds(0, ln)])
        pltpu.sync_copy(
            rows_v.at[pl.ds(0, ln)],
            agg_hbm.at[pl.ds(c * N_PAD + s * rows_per_sub + off, ln)])


_ROW_BLK = 512
_N_ROW_BLKS = NPG // _ROW_BLK


def _mm_body(h_ref, w_ref, o_ref):
    o_ref[...] = jnp.dot(h_ref[...], w_ref[...],
                         preferred_element_type=jnp.float32)


def _tc_matmul(h, w):
    return pl.pallas_call(
        _mm_body,
        grid=(_N_ROW_BLKS,),
        in_specs=[
            pl.BlockSpec((_ROW_BLK, D), lambda i: (i, 0)),
            pl.BlockSpec((D, D), lambda i: (0, 0)),
        ],
        out_specs=pl.BlockSpec((_ROW_BLK, D), lambda i: (i, 0)),
        out_shape=jax.ShapeDtypeStruct((NPG, D), jnp.float32),
    )(h, w)


def _gru_body(with_next, h_ref, agg_ref, wih_ref, whh_ref, bih_ref, bhh_ref,
              wn_ref, h_out_ref, m_out_ref):
    h = h_ref[...]
    gi = jnp.dot(agg_ref[...], wih_ref[...],
                 preferred_element_type=jnp.float32) + bih_ref[...]
    gh = jnp.dot(h, whh_ref[...],
                 preferred_element_type=jnp.float32) + bhh_ref[...]
    r = jax.nn.sigmoid(gi[:, 0:D] + gh[:, 0:D])
    z = jax.nn.sigmoid(gi[:, D:2 * D] + gh[:, D:2 * D])
    n = jnp.tanh(gi[:, 2 * D:3 * D] + r * gh[:, 2 * D:3 * D])
    hn = (1.0 - z) * n + z * h
    h_out_ref[...] = hn
    if with_next:
        m_out_ref[...] = jnp.dot(hn, wn_ref[...],
                                 preferred_element_type=jnp.float32)
    else:
        m_out_ref[...] = hn


def _tc_gru(h, agg, wih_t, whh_t, bih, bhh, w_next, with_next):
    return pl.pallas_call(
        functools.partial(_gru_body, with_next),
        grid=(_N_ROW_BLKS,),
        in_specs=[
            pl.BlockSpec((_ROW_BLK, D), lambda i: (i, 0)),
            pl.BlockSpec((_ROW_BLK, D), lambda i: (i, 0)),
            pl.BlockSpec((D, 3 * D), lambda i: (0, 0)),
            pl.BlockSpec((D, 3 * D), lambda i: (0, 0)),
            pl.BlockSpec((1, 3 * D), lambda i: (0, 0)),
            pl.BlockSpec((1, 3 * D), lambda i: (0, 0)),
            pl.BlockSpec((D, D), lambda i: (0, 0)),
        ],
        out_specs=[
            pl.BlockSpec((_ROW_BLK, D), lambda i: (i, 0)),
            pl.BlockSpec((_ROW_BLK, D), lambda i: (i, 0)),
        ],
        out_shape=[
            jax.ShapeDtypeStruct((NPG, D), jnp.float32),
            jax.ShapeDtypeStruct((NPG, D), jnp.float32),
        ],
    )(h, agg, wih_t, whh_t, bih, bhh, w_next)


def _attn_body(h_ref, gw_ref, gb_ref, o_ref):
    h = h_ref[0]
    gw = gw_ref[...]
    s = jnp.sum(h * gw, axis=1, keepdims=True) + gb_ref[0, 0]
    g = jax.nn.sigmoid(s)
    row = lax.broadcasted_iota(jnp.int32, (N_PAD, 1), 0)
    valid = row < N
    gm = jnp.where(valid, g, -jnp.inf)
    e = jnp.where(valid, jnp.exp(g - jnp.max(gm)), 0.0)
    p = e / jnp.sum(e)
    o_ref[0] = jnp.sum(p * h, axis=0, keepdims=True)


def _tc_attention(h3, gate_w, gate_b):
    return pl.pallas_call(
        _attn_body,
        grid=(2,),
        in_specs=[
            pl.BlockSpec((1, N_PAD, D), lambda g: (g, 0, 0)),
            pl.BlockSpec((1, D), lambda g: (0, 0)),
            pl.BlockSpec((1, 1), lambda g: (0, 0)),
        ],
        out_specs=pl.BlockSpec((1, 1, D), lambda g: (g, 0, 0)),
        out_shape=jax.ShapeDtypeStruct((2, 1, D), jnp.float32),
    )(h3, gate_w, gate_b)


def kernel(x1, x2, edge_index1, edge_index2, edge_attr1, edge_attr2, embed,
           edge_embed, ggnn_w, gru_wih, gru_whh, gru_bih, gru_bhh,
           gate_w, gate_b):
    del edge_attr1, edge_attr2, edge_embed

    i32 = jnp.int32
    zpad_n = jnp.zeros((N_PAD - N,), i32)
    idx_flat = jnp.concatenate(
        [x1[:, 0].astype(i32), zpad_n, x2[:, 0].astype(i32), zpad_n])
    idx2 = idx_flat.reshape(NPG // 128, 128)

    ord1 = jnp.argsort(edge_index1[0])
    ord2 = jnp.argsort(edge_index2[0])
    src1 = edge_index1[0, ord1].astype(i32)
    dst1 = edge_index1[1, ord1].astype(i32)
    src2 = edge_index2[0, ord2].astype(i32)
    dst2 = edge_index2[1, ord2].astype(i32)

    epad_src = jnp.zeros((E_PAD - E,), i32)
    epad_dst = jnp.full((E_PAD - E,), N, i32)
    src_flat = jnp.concatenate([
        src1, epad_src,
        src2 + N_PAD, epad_src + N_PAD,
    ]).reshape(2 * E_PAD // 128, 128)
    dst_flat = jnp.concatenate([
        dst1, epad_dst,
        dst2, epad_dst,
    ]).reshape(2 * E_PAD // 128, 128)

    zeros_rows = jnp.zeros((256, D), jnp.float32)

    wih_t = gru_wih.T
    whh_t = gru_whh.T
    bih = gru_bih.reshape(1, 3 * D)
    bhh = gru_bhh.reshape(1, 3 * D)

    h = _sc_embed_gather(embed, idx2)
    m = _tc_matmul(h, ggnn_w[0])
    for layer in range(L_LAYERS):
        agg = _sc_edge_aggregate(m, src_flat, dst_flat, zeros_rows)
        with_next = layer < L_LAYERS - 1
        w_next = ggnn_w[layer + 1] if with_next else ggnn_w[0]
        h, m = _tc_gru(h, agg, wih_t, whh_t, bih, bhh, w_next, with_next)

    out = _tc_attention(h.reshape(2, N_PAD, D), gate_w, gate_b.reshape(1, 1))
    return (out[0], out[1])


# speedup vs baseline: 1.4014x; 1.4014x over previous
import functools

import jax
import jax.numpy as jnp
from jax import lax
from jax.experimental import pallas as pl
from jax.experimental.pallas import tpu as pltpu
from jax.experimental.pallas import tpu_sc as plsc

N = 10000
E = 320000
D = 128
L_LAYERS = 4

NC = 2
NS = 16
NW = NC * NS

N_PAD = 10240
NPG = 2 * N_PAD
ROWS_PER_W = NPG // NW
K_IDX = 5
CHUNK = K_IDX * 128
E_PAD = 327680
EDGES_PER_SUB = E_PAD // NS
IDX_SLAB = 4
N_SLABS = EDGES_PER_SUB // (IDX_SLAB * 128)
N_SLOTS = 3

_MESH = plsc.VectorSubcoreMesh(
    core_axis_name="c", subcore_axis_name="s", num_cores=NC, num_subcores=NS)


@functools.partial(
    pl.kernel,
    out_type=jax.ShapeDtypeStruct((NPG, D), jnp.float32),
    mesh=_MESH,
    scratch_types=[
        pltpu.VMEM((NPG // 128, 128), jnp.int32),
        pltpu.VMEM((CHUNK, D), jnp.float32),
        pltpu.SemaphoreType.DMA,
    ],
)
def _sc_embed_gather(tab_hbm, idx_hbm, out_hbm, idx_v, rows_v, sem):
    wid = lax.axis_index("s") * NC + lax.axis_index("c")
    pltpu.sync_copy(idx_hbm, idx_v)
    descs = []
    for j in range(K_IDX):
        descs.append(pltpu.async_copy(
            tab_hbm.at[idx_v.at[wid * K_IDX + j]],
            rows_v.at[pl.ds(j * 128, 128)], sem))
    for d in descs:
        d.wait()
    pltpu.sync_copy(rows_v, out_hbm.at[pl.ds(wid * ROWS_PER_W, ROWS_PER_W)])


@functools.partial(
    pl.kernel,
    out_type=jax.ShapeDtypeStruct((NPG, D), jnp.float32),
    mesh=_MESH,
    scratch_types=[
        pltpu.VMEM((IDX_SLAB, 128), jnp.int32),
        pltpu.VMEM((IDX_SLAB, 128), jnp.int32),
        pltpu.VMEM((N_SLOTS * 128, D), jnp.float32),
        pltpu.VMEM_SHARED((N, D), jnp.float32),
        pltpu.SemaphoreType.DMA,
    ],
)
def _sc_edge_aggregate(m_hbm, src_hbm, dst_hbm, zeros_hbm, agg_hbm,
                       src_v, dst_v, rows_v, accum, sem):
    c = lax.axis_index("c")
    s = lax.axis_index("s")
    last = s == NS - 1
    slices_full = ((0, 256), (256, 256), (512, 128))
    slices_last = ((0, 256), (256, 144))
    pltpu.sync_copy(zeros_hbm, rows_v.at[pl.ds(0, 256)])

    @pl.when(jnp.logical_not(last))
    def _():
        for off, ln in slices_full:
            pltpu.sync_copy(rows_v.at[pl.ds(0, ln)],
                            accum.at[pl.ds(s * 640 + off, ln)])

    @pl.when(last)
    def _():
        for off, ln in slices_last:
            pltpu.sync_copy(rows_v.at[pl.ds(0, ln)],
                            accum.at[pl.ds(9600 + off, ln)])

    plsc.subcore_barrier()

    idx_base = c * (E_PAD // 128) + s * (EDGES_PER_SUB // 128)

    def slab_body(t, carry):
        pltpu.sync_copy(src_hbm.at[pl.ds(idx_base + t * IDX_SLAB, IDX_SLAB)],
                        src_v)
        pltpu.sync_copy(dst_hbm.at[pl.ds(idx_base + t * IDX_SLAB, IDX_SLAB)],
                        dst_v)
        descs = [None] * N_SLOTS
        for p in range(N_SLOTS - 1):
            descs[p] = pltpu.async_copy(
                m_hbm.at[src_v.at[p]], rows_v.at[pl.ds(p * 128, 128)], sem)
        for j in range(IDX_SLAB):
            nj = j + N_SLOTS - 1
            if nj < IDX_SLAB:
                descs[nj % N_SLOTS] = pltpu.async_copy(
                    m_hbm.at[src_v.at[nj]],
                    rows_v.at[pl.ds((nj % N_SLOTS) * 128, 128)], sem)
            descs[j % N_SLOTS].wait()
            pltpu.sync_copy(rows_v.at[pl.ds((j % N_SLOTS) * 128, 128)],
                            accum.at[dst_v.at[j]], add=True)
        return carry

    lax.fori_loop(0, N_SLABS, slab_body, 0)
    plsc.subcore_barrier()

    @pl.when(jnp.logical_not(last))
    def _():
        for off, ln in slices_full:
            pltpu.sync_copy(accum.at[pl.ds(s * 640 + off, ln)],
                            rows_v.at[pl.ds(0, ln)])
            pltpu.sync_copy(
                rows_v.at[pl.ds(0, ln)],
                agg_hbm.at[pl.ds(c * N_PAD + s * 640 + off, ln)])

    @pl.when(last)
    def _():
        for off, ln in slices_last:
            pltpu.sync_copy(accum.at[pl.ds(9600 + off, ln)],
                            rows_v.at[pl.ds(0, ln)])
            pltpu.sync_copy(
                rows_v.at[pl.ds(0, ln)],
                agg_hbm.at[pl.ds(c * N_PAD + 9600 + off, ln)])


_ROW_BLK = 512
_N_ROW_BLKS = NPG // _ROW_BLK


def _row_valid_mask():
    rowid = (pl.program_id(0) * _ROW_BLK
             + lax.broadcasted_iota(jnp.int32, (_ROW_BLK, 1), 0))
    return lax.rem(rowid, N_PAD) < N


def _mm_body(h_ref, w_ref, o_ref):
    m = jnp.dot(h_ref[...], w_ref[...], preferred_element_type=jnp.float32)
    o_ref[...] = jnp.where(_row_valid_mask(), m, 0.0)


def _tc_matmul(h, w):
    return pl.pallas_call(
        _mm_body,
        grid=(_N_ROW_BLKS,),
        in_specs=[
            pl.BlockSpec((_ROW_BLK, D), lambda i: (i, 0)),
            pl.BlockSpec((D, D), lambda i: (0, 0)),
        ],
        out_specs=pl.BlockSpec((_ROW_BLK, D), lambda i: (i, 0)),
        out_shape=jax.ShapeDtypeStruct((NPG, D), jnp.float32),
    )(h, w)


def _gru_body(with_next, h_ref, agg_ref, wih_ref, whh_ref, bih_ref, bhh_ref,
              wn_ref, h_out_ref, m_out_ref):
    h = h_ref[...]
    gi = jnp.dot(agg_ref[...], wih_ref[...],
                 preferred_element_type=jnp.float32) + bih_ref[...]
    gh = jnp.dot(h, whh_ref[...],
                 preferred_element_type=jnp.float32) + bhh_ref[...]
    r = jax.nn.sigmoid(gi[:, 0:D] + gh[:, 0:D])
    z = jax.nn.sigmoid(gi[:, D:2 * D] + gh[:, D:2 * D])
    n = jnp.tanh(gi[:, 2 * D:3 * D] + r * gh[:, 2 * D:3 * D])
    hn = (1.0 - z) * n + z * h
    h_out_ref[...] = hn
    if with_next:
        mn = jnp.dot(hn, wn_ref[...], preferred_element_type=jnp.float32)
        m_out_ref[...] = jnp.where(_row_valid_mask(), mn, 0.0)
    else:
        m_out_ref[...] = jnp.where(_row_valid_mask(), hn, 0.0)


def _tc_gru(h, agg, wih_t, whh_t, bih, bhh, w_next, with_next):
    return pl.pallas_call(
        functools.partial(_gru_body, with_next),
        grid=(_N_ROW_BLKS,),
        in_specs=[
            pl.BlockSpec((_ROW_BLK, D), lambda i: (i, 0)),
            pl.BlockSpec((_ROW_BLK, D), lambda i: (i, 0)),
            pl.BlockSpec((D, 3 * D), lambda i: (0, 0)),
            pl.BlockSpec((D, 3 * D), lambda i: (0, 0)),
            pl.BlockSpec((1, 3 * D), lambda i: (0, 0)),
            pl.BlockSpec((1, 3 * D), lambda i: (0, 0)),
            pl.BlockSpec((D, D), lambda i: (0, 0)),
        ],
        out_specs=[
            pl.BlockSpec((_ROW_BLK, D), lambda i: (i, 0)),
            pl.BlockSpec((_ROW_BLK, D), lambda i: (i, 0)),
        ],
        out_shape=[
            jax.ShapeDtypeStruct((NPG, D), jnp.float32),
            jax.ShapeDtypeStruct((NPG, D), jnp.float32),
        ],
    )(h, agg, wih_t, whh_t, bih, bhh, w_next)


def _attn_body(h_ref, gw_ref, gb_ref, o_ref):
    h = h_ref[0]
    gw = gw_ref[...]
    s = jnp.sum(h * gw, axis=1, keepdims=True) + gb_ref[0, 0]
    g = jax.nn.sigmoid(s)
    row = lax.broadcasted_iota(jnp.int32, (N_PAD, 1), 0)
    valid = row < N
    gm = jnp.where(valid, g, -jnp.inf)
    e = jnp.where(valid, jnp.exp(g - jnp.max(gm)), 0.0)
    p = e / jnp.sum(e)
    hm = jnp.where(valid, h, 0.0)
    o_ref[0] = jnp.sum(p * hm, axis=0, keepdims=True)


def _tc_attention(h3, gate_w, gate_b):
    return pl.pallas_call(
        _attn_body,
        grid=(2,),
        in_specs=[
            pl.BlockSpec((1, N_PAD, D), lambda g: (g, 0, 0)),
            pl.BlockSpec((1, D), lambda g: (0, 0)),
            pl.BlockSpec((1, 1), lambda g: (0, 0)),
        ],
        out_specs=pl.BlockSpec((1, 1, D), lambda g: (g, 0, 0)),
        out_shape=jax.ShapeDtypeStruct((2, 1, D), jnp.float32),
    )(h3, gate_w, gate_b)


def kernel(x1, x2, edge_index1, edge_index2, edge_attr1, edge_attr2, embed,
           edge_embed, ggnn_w, gru_wih, gru_whh, gru_bih, gru_bhh,
           gate_w, gate_b):
    del edge_attr1, edge_attr2, edge_embed

    i32 = jnp.int32
    zpad_n = jnp.zeros((N_PAD - N,), i32)
    idx_flat = jnp.concatenate(
        [x1[:, 0].astype(i32), zpad_n, x2[:, 0].astype(i32), zpad_n])
    idx2 = idx_flat.reshape(NPG // 128, 128)

    epad_src = jnp.full((E_PAD - E,), N, i32)
    epad_dst = jnp.zeros((E_PAD - E,), i32)
    src_flat = jnp.concatenate([
        edge_index1[0].astype(i32), epad_src,
        edge_index2[0].astype(i32) + N_PAD, epad_src + N_PAD,
    ]).reshape(2 * E_PAD // 128, 128)
    dst_flat = jnp.concatenate([
        edge_index1[1].astype(i32), epad_dst,
        edge_index2[1].astype(i32), epad_dst,
    ]).reshape(2 * E_PAD // 128, 128)

    zeros_rows = jnp.zeros((256, D), jnp.float32)

    wih_t = gru_wih.T
    whh_t = gru_whh.T
    bih = gru_bih.reshape(1, 3 * D)
    bhh = gru_bhh.reshape(1, 3 * D)

    h = _sc_embed_gather(embed, idx2)
    m = _tc_matmul(h, ggnn_w[0])
    for layer in range(L_LAYERS):
        agg = _sc_edge_aggregate(m, src_flat, dst_flat, zeros_rows)
        with_next = layer < L_LAYERS - 1
        w_next = ggnn_w[layer + 1] if with_next else ggnn_w[0]
        h, m = _tc_gru(h, agg, wih_t, whh_t, bih, bhh, w_next, with_next)

    out = _tc_attention(h.reshape(2, N_PAD, D), gate_w, gate_b.reshape(1, 1))
    return (out[0], out[1])


# speedup vs baseline: 1.4869x; 1.0610x over previous
import functools

import jax
import jax.numpy as jnp
from jax import lax
from jax.experimental import pallas as pl
from jax.experimental.pallas import tpu as pltpu
from jax.experimental.pallas import tpu_sc as plsc

N = 10000
E = 320000
D = 128
L_LAYERS = 4

NC = 2
NS = 16
NW = NC * NS

N_PAD = 10240
NPG = 2 * N_PAD
ROWS_PER_W = NPG // NW
K_IDX = 5
CHUNK = K_IDX * 128
E_PAD = 327680
EDGES_PER_SUB = E_PAD // NS
IDX_SLAB = 32
N_SLABS = EDGES_PER_SUB // (IDX_SLAB * 128)

_MESH = plsc.VectorSubcoreMesh(
    core_axis_name="c", subcore_axis_name="s", num_cores=NC, num_subcores=NS)


@functools.partial(
    pl.kernel,
    out_type=jax.ShapeDtypeStruct((NPG, D), jnp.float32),
    mesh=_MESH,
    scratch_types=[
        pltpu.VMEM((NPG // 128, 128), jnp.int32),
        pltpu.VMEM((CHUNK, D), jnp.float32),
        pltpu.SemaphoreType.DMA,
    ],
)
def _sc_embed_gather(tab_hbm, idx_hbm, out_hbm, idx_v, rows_v, sem):
    wid = lax.axis_index("s") * NC + lax.axis_index("c")
    pltpu.sync_copy(idx_hbm, idx_v)
    descs = []
    for j in range(K_IDX):
        descs.append(pltpu.async_copy(
            tab_hbm.at[idx_v.at[wid * K_IDX + j]],
            rows_v.at[pl.ds(j * 128, 128)], sem))
    for d in descs:
        d.wait()
    pltpu.sync_copy(rows_v, out_hbm.at[pl.ds(wid * ROWS_PER_W, ROWS_PER_W)])


@functools.partial(
    pl.kernel,
    out_type=jax.ShapeDtypeStruct((NPG, D), jnp.float32),
    mesh=_MESH,
    scratch_types=[
        pltpu.VMEM((IDX_SLAB, 128), jnp.int32),
        pltpu.VMEM((IDX_SLAB, 128), jnp.int32),
        pltpu.VMEM((256, D), jnp.float32),
        pltpu.VMEM_SHARED((N_PAD, D), jnp.float32),
        pltpu.SemaphoreType.DMA,
    ],
)
def _sc_edge_aggregate(m_hbm, src_hbm, dst_hbm, zeros_hbm, agg_hbm,
                       src_v, dst_v, rows_v, accum, sem):
    c = lax.axis_index("c")
    s = lax.axis_index("s")
    rows_per_sub = N_PAD // NS
    pltpu.sync_copy(zeros_hbm, rows_v)
    for off, ln in ((0, 256), (256, 256), (512, 128)):
        pltpu.sync_copy(rows_v.at[pl.ds(0, ln)],
                        accum.at[pl.ds(s * rows_per_sub + off, ln)])
    plsc.subcore_barrier()

    idx_base = c * (E_PAD // 128) + s * (EDGES_PER_SUB // 128)

    def slab_body(t, carry):
        pltpu.sync_copy(src_hbm.at[pl.ds(idx_base + t * IDX_SLAB, IDX_SLAB)],
                        src_v)
        pltpu.sync_copy(dst_hbm.at[pl.ds(idx_base + t * IDX_SLAB, IDX_SLAB)],
                        dst_v)
        descs = [None, None]
        descs[0] = pltpu.async_copy(
            m_hbm.at[src_v.at[0]], rows_v.at[pl.ds(0, 128)], sem)
        for j in range(IDX_SLAB):
            if j + 1 < IDX_SLAB:
                descs[(j + 1) % 2] = pltpu.async_copy(
                    m_hbm.at[src_v.at[j + 1]],
                    rows_v.at[pl.ds(((j + 1) % 2) * 128, 128)], sem)
            descs[j % 2].wait()
            pltpu.sync_copy(rows_v.at[pl.ds((j % 2) * 128, 128)],
                            accum.at[dst_v.at[j]], add=True)
        return carry

    lax.fori_loop(0, N_SLABS, slab_body, 0)
    plsc.subcore_barrier()
    for off, ln in ((0, 256), (256, 256), (512, 128)):
        pltpu.sync_copy(accum.at[pl.ds(s * rows_per_sub + off, ln)],
                        rows_v.at[pl.ds(0, ln)])
        pltpu.sync_copy(
            rows_v.at[pl.ds(0, ln)],
            agg_hbm.at[pl.ds(c * N_PAD + s * rows_per_sub + off, ln)])


_ROW_BLK = 512
_N_ROW_BLKS = NPG // _ROW_BLK


def _mm_body(h_ref, w_ref, o_ref):
    o_ref[...] = jnp.dot(h_ref[...], w_ref[...],
                         preferred_element_type=jnp.float32)


def _tc_matmul(h, w):
    return pl.pallas_call(
        _mm_body,
        grid=(_N_ROW_BLKS,),
        in_specs=[
            pl.BlockSpec((_ROW_BLK, D), lambda i: (i, 0)),
            pl.BlockSpec((D, D), lambda i: (0, 0)),
        ],
        out_specs=pl.BlockSpec((_ROW_BLK, D), lambda i: (i, 0)),
        out_shape=jax.ShapeDtypeStruct((NPG, D), jnp.float32),
    )(h, w)


def _gru_body(with_next, h_ref, agg_ref, wih_ref, whh_ref, bih_ref, bhh_ref,
              wn_ref, h_out_ref, m_out_ref):
    h = h_ref[...]
    gi = jnp.dot(agg_ref[...], wih_ref[...],
                 preferred_element_type=jnp.float32) + bih_ref[...]
    gh = jnp.dot(h, whh_ref[...],
                 preferred_element_type=jnp.float32) + bhh_ref[...]
    r = jax.nn.sigmoid(gi[:, 0:D] + gh[:, 0:D])
    z = jax.nn.sigmoid(gi[:, D:2 * D] + gh[:, D:2 * D])
    n = jnp.tanh(gi[:, 2 * D:3 * D] + r * gh[:, 2 * D:3 * D])
    hn = (1.0 - z) * n + z * h
    h_out_ref[...] = hn
    if with_next:
        m_out_ref[...] = jnp.dot(hn, wn_ref[...],
                                 preferred_element_type=jnp.float32)
    else:
        m_out_ref[...] = hn


def _tc_gru(h, agg, wih_t, whh_t, bih, bhh, w_next, with_next):
    return pl.pallas_call(
        functools.partial(_gru_body, with_next),
        grid=(_N_ROW_BLKS,),
        in_specs=[
            pl.BlockSpec((_ROW_BLK, D), lambda i: (i, 0)),
            pl.BlockSpec((_ROW_BLK, D), lambda i: (i, 0)),
            pl.BlockSpec((D, 3 * D), lambda i: (0, 0)),
            pl.BlockSpec((D, 3 * D), lambda i: (0, 0)),
            pl.BlockSpec((1, 3 * D), lambda i: (0, 0)),
            pl.BlockSpec((1, 3 * D), lambda i: (0, 0)),
            pl.BlockSpec((D, D), lambda i: (0, 0)),
        ],
        out_specs=[
            pl.BlockSpec((_ROW_BLK, D), lambda i: (i, 0)),
            pl.BlockSpec((_ROW_BLK, D), lambda i: (i, 0)),
        ],
        out_shape=[
            jax.ShapeDtypeStruct((NPG, D), jnp.float32),
            jax.ShapeDtypeStruct((NPG, D), jnp.float32),
        ],
    )(h, agg, wih_t, whh_t, bih, bhh, w_next)


def _attn_body(h_ref, gw_ref, gb_ref, o_ref):
    h = h_ref[0]
    gw = gw_ref[...]
    s = jnp.sum(h * gw, axis=1, keepdims=True) + gb_ref[0, 0]
    g = jax.nn.sigmoid(s)
    row = lax.broadcasted_iota(jnp.int32, (N_PAD, 1), 0)
    valid = row < N
    gm = jnp.where(valid, g, -jnp.inf)
    e = jnp.where(valid, jnp.exp(g - jnp.max(gm)), 0.0)
    p = e / jnp.sum(e)
    hm = jnp.where(valid, h, 0.0)
    o_ref[0] = jnp.sum(p * hm, axis=0, keepdims=True)


def _tc_attention(h3, gate_w, gate_b):
    return pl.pallas_call(
        _attn_body,
        grid=(2,),
        in_specs=[
            pl.BlockSpec((1, N_PAD, D), lambda g: (g, 0, 0)),
            pl.BlockSpec((1, D), lambda g: (0, 0)),
            pl.BlockSpec((1, 1), lambda g: (0, 0)),
        ],
        out_specs=pl.BlockSpec((1, 1, D), lambda g: (g, 0, 0)),
        out_shape=jax.ShapeDtypeStruct((2, 1, D), jnp.float32),
    )(h3, gate_w, gate_b)


def kernel(x1, x2, edge_index1, edge_index2, edge_attr1, edge_attr2, embed,
           edge_embed, ggnn_w, gru_wih, gru_whh, gru_bih, gru_bhh,
           gate_w, gate_b):
    del edge_attr1, edge_attr2, edge_embed

    i32 = jnp.int32
    zpad_n = jnp.zeros((N_PAD - N,), i32)
    idx_flat = jnp.concatenate(
        [x1[:, 0].astype(i32), zpad_n, x2[:, 0].astype(i32), zpad_n])
    idx2 = idx_flat.reshape(NPG // 128, 128)

    epad_src = jnp.zeros((E_PAD - E,), i32)
    epad_dst = jnp.full((E_PAD - E,), N, i32)
    src_flat = jnp.concatenate([
        edge_index1[0].astype(i32), epad_src,
        edge_index2[0].astype(i32) + N_PAD, epad_src + N_PAD,
    ]).reshape(2 * E_PAD // 128, 128)
    dst_flat = jnp.concatenate([
        edge_index1[1].astype(i32), epad_dst,
        edge_index2[1].astype(i32), epad_dst,
    ]).reshape(2 * E_PAD // 128, 128)

    zeros_rows = jnp.zeros((256, D), jnp.float32)

    wih_t = gru_wih.T
    whh_t = gru_whh.T
    bih = gru_bih.reshape(1, 3 * D)
    bhh = gru_bhh.reshape(1, 3 * D)

    h = _sc_embed_gather(embed, idx2)
    m = _tc_matmul(h, ggnn_w[0])
    for layer in range(L_LAYERS):
        agg = _sc_edge_aggregate(m, src_flat, dst_flat, zeros_rows)
        with_next = layer < L_LAYERS - 1
        w_next = ggnn_w[layer + 1] if with_next else ggnn_w[0]
        h, m = _tc_gru(h, agg, wih_t, whh_t, bih, bhh, w_next, with_next)

    out = _tc_attention(h.reshape(2, N_PAD, D), gate_w, gate_b.reshape(1, 1))
    return (out[0], out[1])
